# baseline (speedup 1.0000x reference)
import functools

import jax
import jax.numpy as jnp
from jax import lax
from jax.experimental import pallas as pl
from jax.experimental.pallas import tpu as pltpu
from jax.experimental.pallas import tpu_sc as plsc

N_CORES = 2
N_SUB = 16
N_NODES = 10000
N_EDGES = 320000
D = 128
DH = D // 2
B = 8
LM_DIM = 1024
MLP_OUT = 256

EPT = N_EDGES // N_SUB
CHUNK = 80
NCHUNK = EPT // CHUNK
HCHUNK = NCHUNK // N_CORES
NBUF = 4
NG = 62
TAIL = NCHUNK - NG * NBUF
ROWS_PER_TILE = N_NODES // N_SUB
SLAB = 125

_MESH = plsc.VectorSubcoreMesh(core_axis_name="c", subcore_axis_name="s")


def _deg_body(dst_hbm, deg_hbm, dst_v, ones_v, zbuf_v, dega_sp, sem):
    c = lax.axis_index("c")
    s = lax.axis_index("s")
    pltpu.sync_copy(dst_hbm.at[s, pl.ds(c * HCHUNK, HCHUNK), :], dst_v)
    zero16 = jnp.zeros((16,), jnp.float32)
    ones16 = jnp.ones((16,), jnp.float32)

    def _fill(i, carry):
        zbuf_v[i, :] = zero16
        return carry

    lax.fori_loop(0, SLAB, _fill, 0)

    def _fill1(i, carry):
        ones_v[i, :] = ones16
        return carry

    lax.fori_loop(0, CHUNK, _fill1, 0)
    for k in range(ROWS_PER_TILE // SLAB):
        pltpu.sync_copy(
            zbuf_v, dega_sp.at[pl.ds(s * ROWS_PER_TILE + k * SLAB, SLAB), :])
    plsc.subcore_barrier()

    def _edge_chunk(j, carry):
        pltpu.async_copy(ones_v, dega_sp.at[dst_v.at[j]], sem, add=True)
        return carry

    lax.fori_loop(0, HCHUNK, _edge_chunk, 0)

    def _drain(j, carry):
        pltpu.make_async_copy(
            deg_hbm.at[0, pl.ds(0, CHUNK), :], ones_v, sem).wait()
        return carry

    lax.fori_loop(0, HCHUNK, _drain, 0)
    plsc.subcore_barrier()
    sl = pl.ds(s * ROWS_PER_TILE, ROWS_PER_TILE)
    pltpu.sync_copy(dega_sp.at[sl, :], deg_hbm.at[c, sl, :])


_deg_kernel = functools.partial(
    pl.kernel,
    out_type=jax.ShapeDtypeStruct((N_CORES, N_NODES, 16), jnp.float32),
    mesh=_MESH,
    compiler_params=pltpu.CompilerParams(use_tc_tiling_on_sc=False),
    scratch_types=[
        pltpu.VMEM((HCHUNK, CHUNK), jnp.int32),
        pltpu.VMEM((CHUNK, 16), jnp.float32),
        pltpu.VMEM((SLAB, 16), jnp.float32),
        pltpu.VMEM_SHARED((N_NODES, 16), jnp.float32),
        pltpu.SemaphoreType.DMA,
    ],
)(_deg_body)


def _scatter_body(ysl_hbm, ysr_hbm, src_hbm, dst_hbm, out_hbm,
                  src_v, dst_v, rows_v, zbuf_v, acc_sp, gsem, ssem):
    c = lax.axis_index("c")
    s = lax.axis_index("s")
    pltpu.sync_copy(src_hbm.at[s], src_v)
    pltpu.sync_copy(dst_hbm.at[s], dst_v)
    zero16 = jnp.zeros((16,), jnp.float32)

    def _zero(i, carry):
        zbuf_v[i // 4, pl.ds((i % 4) * 16, 16)] = zero16
        return carry

    lax.fori_loop(0, SLAB * 4, _zero, 0)
    for k in range(ROWS_PER_TILE // SLAB):
        pltpu.sync_copy(
            zbuf_v, acc_sp.at[pl.ds(s * ROWS_PER_TILE + k * SLAB, SLAB), :])
    plsc.subcore_barrier()

    def _issue_gather(idx, buf):
        @pl.when(c == 0)
        def _gl():
            pltpu.async_copy(ysl_hbm.at[idx], buf, gsem)

        @pl.when(c == 1)
        def _gr():
            pltpu.async_copy(ysr_hbm.at[idx], buf, gsem)

    def _drain(sem):
        pltpu.make_async_copy(
            ysl_hbm.at[pl.ds(0, CHUNK), :], rows_v.at[0, 0], sem).wait()

    for b in range(NBUF):
        _issue_gather(src_v.at[b], rows_v.at[0, b])

    def _group(g, carry):
        p = lax.rem(g, 2)
        q = 1 - p
        for b in range(NBUF):
            _drain(gsem)
        for b in range(NBUF):
            pltpu.async_copy(rows_v.at[p, b],
                             acc_sp.at[dst_v.at[g * NBUF + b]], ssem, add=True)

        @pl.when(g < NG - 1)
        def _next():
            for b in range(NBUF):
                _issue_gather(src_v.at[(g + 1) * NBUF + b], rows_v.at[q, b])

        for b in range(NBUF):
            _drain(ssem)
        return carry

    lax.fori_loop(0, NG, _group, 0)
    for t in range(TAIL):
        _issue_gather(src_v.at[NG * NBUF + t], rows_v.at[0, 0])
        _drain(gsem)
        pltpu.async_copy(rows_v.at[0, 0],
                         acc_sp.at[dst_v.at[NG * NBUF + t]], ssem, add=True)
        _drain(ssem)
    plsc.subcore_barrier()
    for k in range(ROWS_PER_TILE // SLAB):
        sl = pl.ds(s * ROWS_PER_TILE + k * SLAB, SLAB)
        pltpu.sync_copy(acc_sp.at[sl, :], out_hbm.at[c, sl, :])


_scatter_kernel = functools.partial(
    pl.kernel,
    out_type=jax.ShapeDtypeStruct((N_CORES, N_NODES, DH), jnp.float32),
    mesh=_MESH,
    compiler_params=pltpu.CompilerParams(use_tc_tiling_on_sc=False),
    scratch_types=[
        pltpu.VMEM((NCHUNK, CHUNK), jnp.int32),
        pltpu.VMEM((NCHUNK, CHUNK), jnp.int32),
        pltpu.VMEM((2, NBUF, CHUNK, DH), jnp.float32),
        pltpu.VMEM((SLAB, DH), jnp.float32),
        pltpu.VMEM_SHARED((N_NODES, DH), jnp.float32),
        pltpu.SemaphoreType.DMA,
        pltpu.SemaphoreType.DMA,
    ],
)(_scatter_body)


_BN = 2000


def _tc1_body(x_ref, w_ref, degp_ref, ysl_ref, ysr_ref, dinv_ref):
    deg = 1.0 + jnp.sum(degp_ref[...], axis=(0, 2)) * (1.0 / 16.0)
    dinv = lax.rsqrt(deg)
    y = jnp.dot(x_ref[...], w_ref[...], preferred_element_type=jnp.float32)
    ys = y * dinv[:, None]
    ysl_ref[...] = ys[:, :DH]
    ysr_ref[...] = ys[:, DH:]
    dinv_ref[...] = dinv[:, None]


def _tc1(x, w1, deg_partials):
    return pl.pallas_call(
        _tc1_body,
        grid=(N_NODES // _BN,),
        in_specs=[
            pl.BlockSpec((_BN, D), lambda i: (i, 0)),
            pl.BlockSpec((D, D), lambda i: (0, 0)),
            pl.BlockSpec((N_CORES, _BN, 16), lambda i: (0, i, 0)),
        ],
        out_specs=[
            pl.BlockSpec((_BN, DH), lambda i: (i, 0)),
            pl.BlockSpec((_BN, DH), lambda i: (i, 0)),
            pl.BlockSpec((_BN, 1), lambda i: (i, 0)),
        ],
        out_shape=[
            jax.ShapeDtypeStruct((N_NODES, DH), jnp.float32),
            jax.ShapeDtypeStruct((N_NODES, DH), jnp.float32),
            jax.ShapeDtypeStruct((N_NODES, 1), jnp.float32),
        ],
    )(x, w1, deg_partials)


def _tc2_body(acc_ref, ysl_ref, ysr_ref, dinv_ref, b_ref, w_ref,
              ys2l_ref, ys2r_ref):
    dinv = dinv_ref[...]
    agg = jnp.concatenate([acc_ref[0] + ysl_ref[...],
                           acc_ref[1] + ysr_ref[...]], axis=-1)
    tot = agg * dinv + b_ref[...]
    h = jnp.maximum(tot, 0.0)
    y2 = jnp.dot(h, w_ref[...], preferred_element_type=jnp.float32)
    ys2 = y2 * dinv
    ys2l_ref[...] = ys2[:, :DH]
    ys2r_ref[...] = ys2[:, DH:]


def _tc2(acc1, ysl, ysr, dinv, b1, w2):
    return pl.pallas_call(
        _tc2_body,
        grid=(N_NODES // _BN,),
        in_specs=[
            pl.BlockSpec((N_CORES, _BN, DH), lambda i: (0, i, 0)),
            pl.BlockSpec((_BN, DH), lambda i: (i, 0)),
            pl.BlockSpec((_BN, DH), lambda i: (i, 0)),
            pl.BlockSpec((_BN, 1), lambda i: (i, 0)),
            pl.BlockSpec((1, D), lambda i: (0, 0)),
            pl.BlockSpec((D, D), lambda i: (0, 0)),
        ],
        out_specs=[
            pl.BlockSpec((_BN, DH), lambda i: (i, 0)),
            pl.BlockSpec((_BN, DH), lambda i: (i, 0)),
        ],
        out_shape=[
            jax.ShapeDtypeStruct((N_NODES, DH), jnp.float32),
            jax.ShapeDtypeStruct((N_NODES, DH), jnp.float32),
        ],
    )(acc1, ysl, ysr, dinv, b1, w2)


def _tc3_body(acc_ref, ysl_ref, ysr_ref, dinv_ref, b_ref, out_ref):
    agg = jnp.concatenate([acc_ref[0] + ysl_ref[...],
                           acc_ref[1] + ysr_ref[...]], axis=-1)
    out_ref[...] = agg * dinv_ref[...] + b_ref[...]


def _tc3(acc2, ysl, ysr, dinv, b2):
    return pl.pallas_call(
        _tc3_body,
        grid=(N_NODES // _BN,),
        in_specs=[
            pl.BlockSpec((N_CORES, _BN, DH), lambda i: (0, i, 0)),
            pl.BlockSpec((_BN, DH), lambda i: (i, 0)),
            pl.BlockSpec((_BN, DH), lambda i: (i, 0)),
            pl.BlockSpec((_BN, 1), lambda i: (i, 0)),
            pl.BlockSpec((1, D), lambda i: (0, 0)),
        ],
        out_specs=pl.BlockSpec((_BN, D), lambda i: (i, 0)),
        out_shape=jax.ShapeDtypeStruct((N_NODES, D), jnp.float32),
    )(acc2, ysl, ysr, dinv, b2)


_KSUB = 8


def _lm_body(ids_ref, msk_ref, emb_ref, w_ref, b_ref, out_ref):
    i = pl.program_id(0)
    m = msk_ref[i].astype(jnp.float32)
    acc = jnp.zeros((1, MLP_OUT), jnp.float32)
    for k in range(_KSUB):
        a = emb_ref[0, k][None, :]
        acc = acc + jnp.dot(a, w_ref[k], preferred_element_type=jnp.float32)
    out_ref[...] = (m * acc + b_ref[...])[None]


def _lm(ids_col, mask_col, lm_embed, w_mlp, b_mlp):
    emb3 = lm_embed.reshape(-1, _KSUB, LM_DIM // _KSUB)
    w3 = w_mlp.reshape(_KSUB, LM_DIM // _KSUB, MLP_OUT)
    grid_spec = pltpu.PrefetchScalarGridSpec(
        num_scalar_prefetch=2,
        grid=(B,),
        in_specs=[
            pl.BlockSpec((1, _KSUB, LM_DIM // _KSUB),
                         lambda i, ids, msk: (ids[i], 0, 0)),
            pl.BlockSpec((_KSUB, LM_DIM // _KSUB, MLP_OUT),
                         lambda i, ids, msk: (0, 0, 0)),
            pl.BlockSpec((1, MLP_OUT), lambda i, ids, msk: (0, 0)),
        ],
        out_specs=pl.BlockSpec((1, 1, MLP_OUT), lambda i, ids, msk: (i, 0, 0)),
    )
    out = pl.pallas_call(
        _lm_body,
        grid_spec=grid_spec,
        out_shape=jax.ShapeDtypeStruct((B, 1, MLP_OUT), jnp.float32),
    )(ids_col, mask_col, emb3, w3, b_mlp)
    return out.reshape(B, MLP_OUT)


def kernel(ids, mask, edge_index, node_features, lm_embed,
           W_mlp, b_mlp, W1, b1, W2, b2):
    src_r = edge_index[0].reshape(N_SUB, NCHUNK, CHUNK)
    dst_r = edge_index[1].reshape(N_SUB, NCHUNK, CHUNK)

    deg_partials = _deg_kernel(dst_r)
    ys1l, ys1r, dinv = _tc1(node_features, W1, deg_partials)
    acc1 = _scatter_kernel(ys1l, ys1r, src_r, dst_r)
    ys2l, ys2r = _tc2(acc1, ys1l, ys1r, dinv, b1.reshape(1, D), W2)
    acc2 = _scatter_kernel(ys2l, ys2r, src_r, dst_r)
    gcn_out = _tc3(acc2, ys2l, ys2r, dinv, b2.reshape(1, D))

    lm_embeddings = _lm(ids[:, 0], mask[:, 0], lm_embed,
                        W_mlp, b_mlp.reshape(1, -1))
    return (lm_embeddings, gcn_out)


# speedup vs baseline: 1.2235x; 1.2235x over previous
import functools

import jax
import jax.numpy as jnp
from jax import lax
from jax.experimental import pallas as pl
from jax.experimental.pallas import tpu as pltpu
from jax.experimental.pallas import tpu_sc as plsc

N_CORES = 2
N_SUB = 16
N_NODES = 10000
N_EDGES = 320000
D = 128
DH = D // 2
B = 8
LM_DIM = 1024
MLP_OUT = 256

EPT = N_EDGES // N_SUB
CHUNK = 80
NCHUNK = EPT // CHUNK
HCHUNK = NCHUNK // N_CORES
NBUF = 4
NG = 62
TAIL = NCHUNK - NG * NBUF
ROWS_PER_TILE = N_NODES // N_SUB
SLAB = 125

_MESH = plsc.VectorSubcoreMesh(core_axis_name="c", subcore_axis_name="s")


def _deg_body(dst_hbm, deg_hbm, dst_v, ones_v, zbuf_v, dega_sp, sem):
    c = lax.axis_index("c")
    s = lax.axis_index("s")
    pltpu.sync_copy(dst_hbm.at[s, pl.ds(c * HCHUNK, HCHUNK), :], dst_v)
    zero16 = jnp.zeros((16,), jnp.float32)
    ones16 = jnp.ones((16,), jnp.float32)

    def _fill(i, carry):
        zbuf_v[i, :] = zero16
        return carry

    lax.fori_loop(0, SLAB, _fill, 0)

    def _fill1(i, carry):
        ones_v[i, :] = ones16
        return carry

    lax.fori_loop(0, CHUNK, _fill1, 0)
    for k in range(ROWS_PER_TILE // SLAB):
        pltpu.sync_copy(
            zbuf_v, dega_sp.at[pl.ds(s * ROWS_PER_TILE + k * SLAB, SLAB), :])
    plsc.subcore_barrier()

    def _edge_chunk(j, carry):
        pltpu.async_copy(ones_v, dega_sp.at[dst_v.at[j]], sem, add=True)
        return carry

    lax.fori_loop(0, HCHUNK, _edge_chunk, 0)

    def _drain(j, carry):
        pltpu.make_async_copy(
            deg_hbm.at[0, pl.ds(0, CHUNK), :], ones_v, sem).wait()
        return carry

    lax.fori_loop(0, HCHUNK, _drain, 0)
    plsc.subcore_barrier()
    sl = pl.ds(s * ROWS_PER_TILE, ROWS_PER_TILE)
    pltpu.sync_copy(dega_sp.at[sl, :], deg_hbm.at[c, sl, :])


_deg_kernel = functools.partial(
    pl.kernel,
    out_type=jax.ShapeDtypeStruct((N_CORES, N_NODES, 16), jnp.float32),
    mesh=_MESH,
    compiler_params=pltpu.CompilerParams(use_tc_tiling_on_sc=False),
    scratch_types=[
        pltpu.VMEM((HCHUNK, CHUNK), jnp.int32),
        pltpu.VMEM((CHUNK, 16), jnp.float32),
        pltpu.VMEM((SLAB, 16), jnp.float32),
        pltpu.VMEM_SHARED((N_NODES, 16), jnp.float32),
        pltpu.SemaphoreType.DMA,
    ],
)(_deg_body)


def _scatter_body(ysl_hbm, ysr_hbm, src_hbm, dst_hbm, out_hbm,
                  src_v, dst_v, rows_v, zbuf_v, acc_sp, gsem, ssem):
    c = lax.axis_index("c")
    s = lax.axis_index("s")
    pltpu.sync_copy(src_hbm.at[s], src_v)
    pltpu.sync_copy(dst_hbm.at[s], dst_v)
    zero16 = jnp.zeros((16,), jnp.float32)

    def _zero(i, carry):
        zbuf_v[i // 4, pl.ds((i % 4) * 16, 16)] = zero16
        return carry

    lax.fori_loop(0, SLAB * 4, _zero, 0)
    for k in range(ROWS_PER_TILE // SLAB):
        pltpu.sync_copy(
            zbuf_v, acc_sp.at[pl.ds(s * ROWS_PER_TILE + k * SLAB, SLAB), :])
    plsc.subcore_barrier()

    def _issue_gather(idx, buf):
        @pl.when(c == 0)
        def _gl():
            pltpu.async_copy(ysl_hbm.at[idx], buf, gsem)

        @pl.when(c == 1)
        def _gr():
            pltpu.async_copy(ysr_hbm.at[idx], buf, gsem)

    def _drain(sem):
        pltpu.make_async_copy(
            ysl_hbm.at[pl.ds(0, CHUNK), :], rows_v.at[0, 0], sem).wait()

    for b in range(NBUF):
        _issue_gather(src_v.at[b], rows_v.at[0, b])

    def _group(g, carry):
        p = lax.rem(g, 2)
        q = 1 - p
        for b in range(NBUF):
            _drain(gsem)
        for b in range(NBUF):
            pltpu.async_copy(rows_v.at[p, b],
                             acc_sp.at[dst_v.at[g * NBUF + b]], ssem, add=True)

        @pl.when(g < NG - 1)
        def _next():
            for b in range(NBUF):
                _issue_gather(src_v.at[(g + 1) * NBUF + b], rows_v.at[q, b])

        for b in range(NBUF):
            _drain(ssem)
        return carry

    lax.fori_loop(0, NG, _group, 0)
    for t in range(TAIL):
        _issue_gather(src_v.at[NG * NBUF + t], rows_v.at[0, 0])
        _drain(gsem)
        pltpu.async_copy(rows_v.at[0, 0],
                         acc_sp.at[dst_v.at[NG * NBUF + t]], ssem, add=True)
        _drain(ssem)
    plsc.subcore_barrier()
    for k in range(ROWS_PER_TILE // SLAB):
        sl = pl.ds(s * ROWS_PER_TILE + k * SLAB, SLAB)
        pltpu.sync_copy(acc_sp.at[sl, :], out_hbm.at[c, sl, :])


_scatter_kernel = functools.partial(
    pl.kernel,
    out_type=jax.ShapeDtypeStruct((N_CORES, N_NODES, DH), jnp.float32),
    mesh=_MESH,
    compiler_params=pltpu.CompilerParams(use_tc_tiling_on_sc=False),
    scratch_types=[
        pltpu.VMEM((NCHUNK, CHUNK), jnp.int32),
        pltpu.VMEM((NCHUNK, CHUNK), jnp.int32),
        pltpu.VMEM((2, NBUF, CHUNK, DH), jnp.float32),
        pltpu.VMEM((SLAB, DH), jnp.float32),
        pltpu.VMEM_SHARED((N_NODES, DH), jnp.float32),
        pltpu.SemaphoreType.DMA,
        pltpu.SemaphoreType.DMA,
    ],
)(_scatter_body)


_BN = 2000


def _tc1_body(x_ref, w_ref, degp_ref, ysl_ref, ysr_ref, dinv_ref):
    deg = 1.0 + jnp.sum(degp_ref[...], axis=(0, 2)) * (1.0 / 16.0)
    dinv = lax.rsqrt(deg)
    y = jnp.dot(x_ref[...], w_ref[...], preferred_element_type=jnp.float32)
    ys = y * dinv[:, None]
    ysl_ref[...] = ys[:, :DH]
    ysr_ref[...] = ys[:, DH:]
    dinv_ref[...] = dinv[:, None]


def _tc1(x, w1, deg_partials):
    return pl.pallas_call(
        _tc1_body,
        grid=(N_NODES // _BN,),
        in_specs=[
            pl.BlockSpec((_BN, D), lambda i: (i, 0)),
            pl.BlockSpec((D, D), lambda i: (0, 0)),
            pl.BlockSpec((N_CORES, _BN, 16), lambda i: (0, i, 0)),
        ],
        out_specs=[
            pl.BlockSpec((_BN, DH), lambda i: (i, 0)),
            pl.BlockSpec((_BN, DH), lambda i: (i, 0)),
            pl.BlockSpec((_BN, 1), lambda i: (i, 0)),
        ],
        out_shape=[
            jax.ShapeDtypeStruct((N_NODES, DH), jnp.float32),
            jax.ShapeDtypeStruct((N_NODES, DH), jnp.float32),
            jax.ShapeDtypeStruct((N_NODES, 1), jnp.float32),
        ],
    )(x, w1, deg_partials)


def _tc2_body(acc_ref, ysl_ref, ysr_ref, dinv_ref, b_ref, w_ref,
              ys2l_ref, ys2r_ref):
    dinv = dinv_ref[...]
    agg = jnp.concatenate([acc_ref[0] + ysl_ref[...],
                           acc_ref[1] + ysr_ref[...]], axis=-1)
    tot = agg * dinv + b_ref[...]
    h = jnp.maximum(tot, 0.0)
    y2 = jnp.dot(h, w_ref[...], preferred_element_type=jnp.float32)
    ys2 = y2 * dinv
    ys2l_ref[...] = ys2[:, :DH]
    ys2r_ref[...] = ys2[:, DH:]


def _tc2(acc1, ysl, ysr, dinv, b1, w2):
    return pl.pallas_call(
        _tc2_body,
        grid=(N_NODES // _BN,),
        in_specs=[
            pl.BlockSpec((N_CORES, _BN, DH), lambda i: (0, i, 0)),
            pl.BlockSpec((_BN, DH), lambda i: (i, 0)),
            pl.BlockSpec((_BN, DH), lambda i: (i, 0)),
            pl.BlockSpec((_BN, 1), lambda i: (i, 0)),
            pl.BlockSpec((1, D), lambda i: (0, 0)),
            pl.BlockSpec((D, D), lambda i: (0, 0)),
        ],
        out_specs=[
            pl.BlockSpec((_BN, DH), lambda i: (i, 0)),
            pl.BlockSpec((_BN, DH), lambda i: (i, 0)),
        ],
        out_shape=[
            jax.ShapeDtypeStruct((N_NODES, DH), jnp.float32),
            jax.ShapeDtypeStruct((N_NODES, DH), jnp.float32),
        ],
    )(acc1, ysl, ysr, dinv, b1, w2)


def _tc3_body(acc_ref, ysl_ref, ysr_ref, dinv_ref, b_ref, out_ref):
    agg = jnp.concatenate([acc_ref[0] + ysl_ref[...],
                           acc_ref[1] + ysr_ref[...]], axis=-1)
    out_ref[...] = agg * dinv_ref[...] + b_ref[...]


def _tc3(acc2, ysl, ysr, dinv, b2):
    return pl.pallas_call(
        _tc3_body,
        grid=(N_NODES // _BN,),
        in_specs=[
            pl.BlockSpec((N_CORES, _BN, DH), lambda i: (0, i, 0)),
            pl.BlockSpec((_BN, DH), lambda i: (i, 0)),
            pl.BlockSpec((_BN, DH), lambda i: (i, 0)),
            pl.BlockSpec((_BN, 1), lambda i: (i, 0)),
            pl.BlockSpec((1, D), lambda i: (0, 0)),
        ],
        out_specs=pl.BlockSpec((_BN, D), lambda i: (i, 0)),
        out_shape=jax.ShapeDtypeStruct((N_NODES, D), jnp.float32),
    )(acc2, ysl, ysr, dinv, b2)


def _lm_body(ids_ref, msk_ref, emb_ref, w_ref, b_ref, out_ref):
    i = pl.program_id(0)
    idx = ids_ref[i]
    sub = lax.rem(idx, 8)
    m = msk_ref[i].astype(jnp.float32)
    blk = emb_ref[...]
    sel = (lax.broadcasted_iota(jnp.int32, (8, 1), 0) == sub).astype(
        jnp.float32)
    row = jnp.sum(blk * sel, axis=0, keepdims=True)
    y = jnp.dot(row, w_ref[...], preferred_element_type=jnp.float32)
    out_ref[...] = (m * y + b_ref[...])[None]


def _lm(ids_col, mask_col, lm_embed, w_mlp, b_mlp):
    grid_spec = pltpu.PrefetchScalarGridSpec(
        num_scalar_prefetch=2,
        grid=(B,),
        in_specs=[
            pl.BlockSpec((8, LM_DIM), lambda i, ids, msk: (ids[i] // 8, 0)),
            pl.BlockSpec((LM_DIM, MLP_OUT), lambda i, ids, msk: (0, 0)),
            pl.BlockSpec((1, MLP_OUT), lambda i, ids, msk: (0, 0)),
        ],
        out_specs=pl.BlockSpec((1, 1, MLP_OUT), lambda i, ids, msk: (i, 0, 0)),
    )
    out = pl.pallas_call(
        _lm_body,
        grid_spec=grid_spec,
        out_shape=jax.ShapeDtypeStruct((B, 1, MLP_OUT), jnp.float32),
    )(ids_col, mask_col, lm_embed, w_mlp, b_mlp)
    return out.reshape(B, MLP_OUT)


def kernel(ids, mask, edge_index, node_features, lm_embed,
           W_mlp, b_mlp, W1, b1, W2, b2):
    src_r = edge_index[0].reshape(N_SUB, NCHUNK, CHUNK)
    dst_r = edge_index[1].reshape(N_SUB, NCHUNK, CHUNK)

    deg_partials = _deg_kernel(dst_r)
    ys1l, ys1r, dinv = _tc1(node_features, W1, deg_partials)
    acc1 = _scatter_kernel(ys1l, ys1r, src_r, dst_r)
    ys2l, ys2r = _tc2(acc1, ys1l, ys1r, dinv, b1.reshape(1, D), W2)
    acc2 = _scatter_kernel(ys2l, ys2r, src_r, dst_r)
    gcn_out = _tc3(acc2, ys2l, ys2r, dinv, b2.reshape(1, D))

    lm_embeddings = _lm(ids[:, 0], mask[:, 0], lm_embed,
                        W_mlp, b_mlp.reshape(1, -1))
    return (lm_embeddings, gcn_out)
